# baseline (speedup 1.0000x reference)
import functools

import jax
import jax.numpy as jnp
from jax.experimental import pallas as pl
from jax.experimental.pallas import tpu as pltpu


def _index_points(points, idx):
    return jax.vmap(lambda p, i: p[i])(points, idx)


def _square_distance(src, dst):
    return jnp.sum((src[:, :, None, :] - dst[:, None, :, :]) ** 2, axis=-1)


def _farthest_point_sample(xyz, npoint):
    b, n, _ = xyz.shape
    def body(i, state):
        centroids, distance, farthest = state
        centroids = centroids.at[:, i].set(farthest)
        centroid = jnp.take_along_axis(xyz, farthest[:, None, None].astype(jnp.int32), axis=1)
        dist = jnp.sum((xyz - centroid) ** 2, axis=-1)
        distance = jnp.minimum(distance, dist)
        farthest = jnp.argmax(distance, axis=-1).astype(jnp.int32)
        return centroids, distance, farthest
    init = (jnp.zeros((b, npoint), dtype=jnp.int32),
            jnp.full((b, n), 1e10, dtype=xyz.dtype),
            jnp.zeros((b,), dtype=jnp.int32))
    centroids, _, _ = jax.lax.fori_loop(0, npoint, body, init)
    return centroids


def _query_ball_point(radius, nsample, xyz, new_xyz):
    b, n, _ = xyz.shape
    s = new_xyz.shape[1]
    sqr = _square_distance(new_xyz, xyz)
    gidx = jnp.broadcast_to(jnp.arange(n, dtype=jnp.int32), (b, s, n))
    gidx = jnp.where(sqr > radius ** 2, n, gidx)
    gidx = jnp.sort(gidx, axis=-1)[:, :, :nsample]
    first = gidx[:, :, 0:1]
    gidx = jnp.where(gidx == n, first, gidx)
    return jnp.clip(gidx, 0, n - 1)


def _bn(x, gamma, beta, axes):
    mean = jnp.mean(x, axis=axes, keepdims=True)
    var = jnp.var(x, axis=axes, keepdims=True)
    return (x - mean) / jnp.sqrt(var + 1e-5) * gamma + beta


def _sa_layer(xyz, points, mlp_params, npoint, radius, nsample, group_all):
    if group_all:
        new_xyz = jnp.zeros((xyz.shape[0], 1, 3), dtype=xyz.dtype)
        x = xyz[:, None, :, :]
        if points is not None:
            x = jnp.concatenate([x, points[:, None, :, :]], axis=-1)
    else:
        fps_idx = _farthest_point_sample(xyz, npoint)
        new_xyz = _index_points(xyz, fps_idx)
        idx = _query_ball_point(radius, nsample, xyz, new_xyz)
        grouped_xyz = _index_points(xyz, idx) - new_xyz[:, :, None, :]
        if points is not None:
            grouped_pts = _index_points(points, idx)
            x = jnp.concatenate([grouped_xyz, grouped_pts], axis=-1)
        else:
            x = grouped_xyz
    for layer in mlp_params:
        x = jnp.einsum('bskc,oc->bsko', x, layer['W']) + layer['b']
        x = _bn(x, layer['gamma'], layer['beta'], (0, 1, 2))
        x = jax.nn.relu(x)
    return new_xyz, jnp.max(x, axis=2)


def _fc_head_kernel(x_ref, w1_ref, b1_ref, g1_ref, bb1_ref,
                    w2_ref, b2_ref, g2_ref, bb2_ref,
                    w3_ref, b3_ref, out_ref):
    x = x_ref[...]

    def bn_rows(y, g, b):
        mean = jnp.mean(y, axis=0, keepdims=True)
        var = jnp.mean((y - mean) ** 2, axis=0, keepdims=True)
        return (y - mean) / jnp.sqrt(var + 1e-5) * g + b

    y = jnp.dot(x, w1_ref[...], preferred_element_type=jnp.float32) + b1_ref[...]
    y = jax.nn.relu(bn_rows(y, g1_ref[...], bb1_ref[...]))
    y = jnp.dot(y, w2_ref[...], preferred_element_type=jnp.float32) + b2_ref[...]
    y = jax.nn.relu(bn_rows(y, g2_ref[...], bb2_ref[...]))
    out_ref[...] = (jnp.dot(y, w3_ref[...], preferred_element_type=jnp.float32)
                    + b3_ref[...])


def _fc_head(x, params):
    b = x.shape[0]
    args = (x,
            params['fc1_W'].T, params['fc1_b'][None, :],
            params['bn1_g'][None, :], params['bn1_b'][None, :],
            params['fc2_W'].T, params['fc2_b'][None, :],
            params['bn2_g'][None, :], params['bn2_b'][None, :],
            params['fc3_W'].T, params['fc3_b'][None, :])
    return pl.pallas_call(
        _fc_head_kernel,
        out_shape=jax.ShapeDtypeStruct((b, params['fc3_W'].shape[0]), jnp.float32),
    )(*args)


def kernel(xyz, params):
    pts = xyz[..., :3]
    norm = xyz[..., 3:]
    l1_xyz, l1_pts = _sa_layer(pts, norm, params['sa1'], 512, 0.2, 32, False)
    l2_xyz, l2_pts = _sa_layer(l1_xyz, l1_pts, params['sa2'], 128, 0.4, 64, False)
    _, l3_pts = _sa_layer(l2_xyz, l2_pts, params['sa3'], None, None, None, True)
    x = l3_pts.reshape(xyz.shape[0], 1024)
    return _fc_head(x, params)


# speedup vs baseline: 1.1597x; 1.1597x over previous
import functools

import jax
import jax.numpy as jnp
from jax.experimental import pallas as pl
from jax.experimental.pallas import tpu as pltpu


def _fps_kernel(x_ref, y_ref, z_ref, cx_ref, cy_ref, cz_ref, *, npoint,
                variant=3):
    b, n = x_ref.shape
    x = x_ref[...]
    y = y_ref[...]
    z = z_ref[...]
    iota = jax.lax.broadcasted_iota(jnp.int32, (b, n), 1)

    def body(i, state):
        distance, farthest = state
        mask = iota == farthest
        cx = jnp.sum(jnp.where(mask, x, 0.0), axis=1, keepdims=True)
        cy = jnp.sum(jnp.where(mask, y, 0.0), axis=1, keepdims=True)
        cz = jnp.sum(jnp.where(mask, z, 0.0), axis=1, keepdims=True)
        cx_ref[pl.ds(i, 1)] = cx[None]
        cy_ref[pl.ds(i, 1)] = cy[None]
        cz_ref[pl.ds(i, 1)] = cz[None]
        dx = x - cx
        dy = y - cy
        dz = z - cz
        if variant == 0:
            dist = (dx * dx + dy * dy) + dz * dz
        elif variant == 1:
            dist = dx * dx + (dy * dy + dz * dz)
        elif variant == 2:
            dist = (dz * dz + dy * dy) + dx * dx
        elif variant == 3:
            dist = (dx * dx + dz * dz) + dy * dy
        distance = jnp.minimum(distance, dist)
        m = jnp.max(distance, axis=1, keepdims=True)
        farthest = jnp.min(jnp.where(distance == m, iota, n), axis=1,
                           keepdims=True)
        return distance, farthest

    init = (jnp.full((b, n), 1e10, dtype=jnp.float32),
            jnp.zeros((b, 1), dtype=jnp.int32))
    jax.lax.fori_loop(0, npoint, body, init)


def _fps(x, y, z, npoint, variant=3):
    b, n = x.shape
    out = jax.ShapeDtypeStruct((npoint, b, 1), jnp.float32)
    cx, cy, cz = pl.pallas_call(
        functools.partial(_fps_kernel, npoint=npoint, variant=variant),
        out_shape=(out, out, out),
    )(x, y, z)
    return (cx[..., 0].T, cy[..., 0].T, cz[..., 0].T)


def _ballquery_kernel(x_ref, y_ref, z_ref, nx_ref, ny_ref, nz_ref, idx_ref,
                      *, radius, nsample):
    n = x_ref.shape[2]
    s = nx_ref.shape[1]
    x = x_ref[0]
    y = y_ref[0]
    z = z_ref[0]
    nx = nx_ref[0]
    ny = ny_ref[0]
    nz = nz_ref[0]
    dx = nx - x
    dy = ny - y
    dz = nz - z
    sqr = (dx * dx + dy * dy) + dz * dz
    mask = sqr <= radius * radius
    c = mask.astype(jnp.int32)
    sh = 1
    while sh < n:
        zpad = jnp.zeros((s, sh), dtype=jnp.int32)
        c = c + jnp.concatenate([zpad, c[:, : n - sh]], axis=1)
        sh *= 2
    count = c[:, n - 1:n]
    iota = jax.lax.broadcasted_iota(jnp.int32, (s, n), 1)
    big = jnp.where(mask, iota, n)
    cols = []
    idx0 = jnp.min(jnp.where(c == 1, big, n), axis=1, keepdims=True)
    for k in range(nsample):
        if k == 0:
            idxk = idx0
        else:
            idxk = jnp.min(jnp.where(c == k + 1, big, n), axis=1,
                           keepdims=True)
            idxk = jnp.where(count > k, idxk, idx0)
        cols.append(idxk)
    out = jnp.concatenate(cols, axis=1)
    idx_ref[0] = jnp.clip(out, 0, n - 1)


def _query_ball(radius, nsample, x, y, z, nx, ny, nz):
    b, n = x.shape
    s = nx.shape[1]
    pts_spec = pl.BlockSpec((1, 1, n), lambda i: (i, 0, 0))
    ctr_spec = pl.BlockSpec((1, s, 1), lambda i: (i, 0, 0))
    return pl.pallas_call(
        functools.partial(_ballquery_kernel, radius=radius, nsample=nsample),
        grid=(b,),
        in_specs=[pts_spec] * 3 + [ctr_spec] * 3,
        out_specs=pl.BlockSpec((1, s, nsample), lambda i: (i, 0, 0)),
        out_shape=jax.ShapeDtypeStruct((b, s, nsample), jnp.int32),
    )(x.reshape(b, 1, n), y.reshape(b, 1, n), z.reshape(b, 1, n),
      nx.reshape(b, s, 1), ny.reshape(b, s, 1), nz.reshape(b, s, 1))


def _index_points(points, idx):
    return jax.vmap(lambda p, i: p[i])(points, idx)


def _bn(x, gamma, beta, axes):
    mean = jnp.mean(x, axis=axes, keepdims=True)
    var = jnp.var(x, axis=axes, keepdims=True)
    return (x - mean) / jnp.sqrt(var + 1e-5) * gamma + beta


def _mlp_pool(x, mlp_params):
    for layer in mlp_params:
        x = jnp.einsum('bskc,oc->bsko', x, layer['W']) + layer['b']
        x = _bn(x, layer['gamma'], layer['beta'], (0, 1, 2))
        x = jax.nn.relu(x)
    return jnp.max(x, axis=2)


def _sa_layer(xyz_planes, points, mlp_params, npoint, radius, nsample):
    x, y, z = xyz_planes
    nx, ny, nz = _fps(x, y, z, npoint)
    idx = _query_ball(radius, nsample, x, y, z, nx, ny, nz)
    xyz = jnp.stack([x, y, z], axis=-1)
    new_xyz = jnp.stack([nx, ny, nz], axis=-1)
    grouped_xyz = _index_points(xyz, idx) - new_xyz[:, :, None, :]
    grouped_pts = _index_points(points, idx)
    feat = jnp.concatenate([grouped_xyz, grouped_pts], axis=-1)
    return (nx, ny, nz), _mlp_pool(feat, mlp_params)


def _fc_head_kernel(x_ref, w1_ref, b1_ref, g1_ref, bb1_ref,
                    w2_ref, b2_ref, g2_ref, bb2_ref,
                    w3_ref, b3_ref, out_ref):
    x = x_ref[...]

    def bn_rows(yv, g, b):
        mean = jnp.mean(yv, axis=0, keepdims=True)
        var = jnp.mean((yv - mean) ** 2, axis=0, keepdims=True)
        return (yv - mean) / jnp.sqrt(var + 1e-5) * g + b

    yv = jnp.dot(x, w1_ref[...], preferred_element_type=jnp.float32) + b1_ref[...]
    yv = jax.nn.relu(bn_rows(yv, g1_ref[...], bb1_ref[...]))
    yv = jnp.dot(yv, w2_ref[...], preferred_element_type=jnp.float32) + b2_ref[...]
    yv = jax.nn.relu(bn_rows(yv, g2_ref[...], bb2_ref[...]))
    out_ref[...] = (jnp.dot(yv, w3_ref[...], preferred_element_type=jnp.float32)
                    + b3_ref[...])


def _fc_head(x, params):
    b = x.shape[0]
    args = (x,
            params['fc1_W'].T, params['fc1_b'][None, :],
            params['bn1_g'][None, :], params['bn1_b'][None, :],
            params['fc2_W'].T, params['fc2_b'][None, :],
            params['bn2_g'][None, :], params['bn2_b'][None, :],
            params['fc3_W'].T, params['fc3_b'][None, :])
    return pl.pallas_call(
        _fc_head_kernel,
        out_shape=jax.ShapeDtypeStruct((b, params['fc3_W'].shape[0]), jnp.float32),
    )(*args)


def kernel(xyz, params):
    pts = xyz[..., :3]
    norm = xyz[..., 3:]
    planes0 = (pts[..., 0], pts[..., 1], pts[..., 2])
    l1_planes, l1_pts = _sa_layer(planes0, norm, params['sa1'], 512, 0.2, 32)
    l2_planes, l2_pts = _sa_layer(l1_planes, l1_pts, params['sa2'], 128, 0.4, 64)
    l2_xyz = jnp.stack(l2_planes, axis=-1)
    x = jnp.concatenate([l2_xyz, l2_pts], axis=-1)[:, None, :, :]
    x = _mlp_pool(x, params['sa3'])
    x = x.reshape(xyz.shape[0], 1024)
    return _fc_head(x, params)
